# baseline (speedup 1.0000x reference)
import functools

import jax
import jax.numpy as jnp
from jax import lax
from jax.experimental import pallas as pl
from jax.experimental.pallas import tpu as pltpu
from jax.experimental.pallas import tpu_sc as plsc

POS_LENGTH = 8
PI = 3.141592653589793

_ROWS_PER_W = 13312
_SLICES = _ROWS_PER_W // 128
_FIRE = 13


def _sc_gather_body(tbl_hbm, gidx_hbm, ox_hbm, oy_hbm,
                    idx_v, idy_v, vx, vy, sem):
    wid = lax.axis_index("s") * 2 + lax.axis_index("c")
    base = wid * _ROWS_PER_W
    pltpu.sync_copy(gidx_hbm.at[0].at[pl.ds(wid * _SLICES, _SLICES)], idx_v)
    pltpu.sync_copy(gidx_hbm.at[1].at[pl.ds(wid * _SLICES, _SLICES)], idy_v)

    def grp(g):
        cps = []
        for j in range(_FIRE):
            i = g * _FIRE + j
            cps.append(pltpu.async_copy(
                tbl_hbm.at[idx_v.at[i]], vx.at[pl.ds(i * 128, 128)], sem))
            cps.append(pltpu.async_copy(
                tbl_hbm.at[idy_v.at[i]], vy.at[pl.ds(i * 128, 128)], sem))
        for cp in cps:
            cp.wait()

    pl.loop(0, _SLICES // _FIRE)(grp)
    pltpu.sync_copy(vx, ox_hbm.at[pl.ds(base, _ROWS_PER_W)])
    pltpu.sync_copy(vy, oy_hbm.at[pl.ds(base, _ROWS_PER_W)])


def _sc_gather(mesh_1d, gidx2):
    n_pad = gidx2.shape[1] * 128
    mesh = plsc.VectorSubcoreMesh(core_axis_name="c", subcore_axis_name="s")
    k = functools.partial(
        pl.kernel,
        mesh=mesh,
        out_type=[jax.ShapeDtypeStruct((n_pad,), jnp.float32),
                  jax.ShapeDtypeStruct((n_pad,), jnp.float32)],
        scratch_types=[
            pltpu.VMEM((_SLICES, 128), jnp.int32),
            pltpu.VMEM((_SLICES, 128), jnp.int32),
            pltpu.VMEM((_ROWS_PER_W,), jnp.float32),
            pltpu.VMEM((_ROWS_PER_W,), jnp.float32),
            pltpu.SemaphoreType.DMA,
        ],
    )(_sc_gather_body)
    return k(mesh_1d, gidx2)


def _centers_body(mpg_ref, msk_ref, c_ref):
    v = mpg_ref[...]
    m = msk_ref[...]
    lane = lax.broadcasted_iota(jnp.int32, (64, 2), 0)
    col = lax.broadcasted_iota(jnp.int32, (64, 2), 1)
    sel = ((lane % 2) == col).astype(jnp.float32)
    s = jnp.dot(v, sel, preferred_element_type=jnp.float32,
                precision=lax.Precision.HIGHEST)
    den = jnp.sum(m, axis=-1, keepdims=True) + 1e-8
    c_ref[...] = s / den


def _tc_centers(mp_g, cluster_mask):
    B, K, C = cluster_mask.shape
    mpg = mp_g.reshape(B * K, 2 * C)
    msk = cluster_mask.reshape(B * K, C)
    out = pl.pallas_call(
        _centers_body,
        out_shape=jax.ShapeDtypeStruct((B * K, 2), jnp.float32),
    )(mpg, msk)
    return out.reshape(B, K, 2)


def _emb32(p2):
    rows = p2.shape[0]
    p32 = jnp.repeat(p2, 16, axis=1)
    lane = lax.broadcasted_iota(jnp.int32, (rows, 32), 1)
    freq = (1 << (lane % 8)).astype(jnp.float32) * PI
    phase = jnp.where((lane % 16) >= 8, 0.5 * PI, 0.0)
    return jnp.cos(p32 * freq - phase)


def _pe_body(a_ref, b_ref, o_ref):
    o_ref[...] = _emb32(a_ref[...] - b_ref[...])


def _tc_pe(crep, mp_g_flat):
    n = crep.shape[0]
    rb = 8000
    return pl.pallas_call(
        _pe_body,
        grid=(n // rb,),
        in_specs=[pl.BlockSpec((rb, 2), lambda i: (i, 0)),
                  pl.BlockSpec((rb, 2), lambda i: (i, 0))],
        out_specs=pl.BlockSpec((rb, 32), lambda i: (i, 0)),
        out_shape=jax.ShapeDtypeStruct((n, 32), jnp.float32),
    )(crep, mp_g_flat)


def _out1_body(mesh_ref, rel_ref, o_ref):
    o_ref[...] = jnp.concatenate(
        [_emb32(mesh_ref[...]), rel_ref[...]], axis=-1)


def _tc_out1(mesh_flat, rel_flat):
    n = mesh_flat.shape[0]
    rb = 8000
    return pl.pallas_call(
        _out1_body,
        grid=(n // rb,),
        in_specs=[pl.BlockSpec((rb, 2), lambda i: (i, 0)),
                  pl.BlockSpec((rb, 32), lambda i: (i, 0))],
        out_specs=pl.BlockSpec((rb, 64), lambda i: (i, 0)),
        out_shape=jax.ShapeDtypeStruct((n, 64), jnp.float32),
    )(mesh_flat, rel_flat)


def _out2_body(c_ref, o_ref):
    o_ref[...] = _emb32(c_ref[...])


def _tc_out2(centers_flat):
    n = centers_flat.shape[0]
    return pl.pallas_call(
        _out2_body,
        grid=(1,),
        in_specs=[pl.BlockSpec((n, 2), lambda i: (0, 0))],
        out_specs=pl.BlockSpec((n, 32), lambda i: (0, 0)),
        out_shape=jax.ShapeDtypeStruct((n, 32), jnp.float32),
    )(centers_flat)


def kernel(mesh_pos, clusters, cluster_mask):
    B, N, _ = mesh_pos.shape
    _, K, C = clusters.shape
    idx = clusters.reshape(B, K * C)

    gidx = (idx + (jnp.arange(B, dtype=jnp.int32) * N)[:, None]).reshape(-1)
    n_pad = 32 * _ROWS_PER_W
    gidx_pad = jnp.concatenate(
        [gidx * 2, jnp.zeros((n_pad - gidx.shape[0],), jnp.int32)])
    gidx2 = jnp.stack([gidx_pad, gidx_pad + 1]).reshape(2, 32 * _SLICES, 128)
    mesh_1d = mesh_pos.reshape(2 * B * N)
    gx, gy = _sc_gather(mesh_1d, gidx2)
    mp_g_flat = jnp.stack([gx[:B * N], gy[:B * N]], axis=-1)
    mesh_flat = mesh_pos.reshape(B * N, 2)
    mp_g = mp_g_flat.reshape(B, K * C, 2)

    centers = _tc_centers(mp_g, cluster_mask)

    crep = jnp.repeat(centers, C, axis=1).reshape(B * K * C, 2)
    pe = _tc_pe(crep, mp_g_flat).reshape(B, K * C, 32)

    relative = jax.vmap(lambda base, i, src: base.at[i].set(src))(pe, idx, pe)
    relative = relative[:, :N]

    out1 = _tc_out1(mesh_flat, relative.reshape(B * N, 32)).reshape(B, N, 64)
    out2 = _tc_out2(centers.reshape(B * K, 2)).reshape(B, K, 32)
    return out1, out2


# speedup vs baseline: 1.1199x; 1.1199x over previous
import functools

import jax
import jax.numpy as jnp
from jax import lax
from jax.experimental import pallas as pl
from jax.experimental.pallas import tpu as pltpu
from jax.experimental.pallas import tpu_sc as plsc

POS_LENGTH = 8
PI = 3.141592653589793

_ROWS_PER_W = 13312
_SLICES = _ROWS_PER_W // 128
_FIRE = 13


def _sc_gather_body(tbl_hbm, gidx_hbm, ox_hbm, oy_hbm,
                    idx_v, idy_v, vx, vy, sem):
    wid = lax.axis_index("s") * 2 + lax.axis_index("c")
    base = wid * _ROWS_PER_W
    pltpu.sync_copy(gidx_hbm.at[0].at[pl.ds(wid * _SLICES, _SLICES)], idx_v)
    pltpu.sync_copy(gidx_hbm.at[1].at[pl.ds(wid * _SLICES, _SLICES)], idy_v)

    def grp(g):
        cps = []
        for j in range(_FIRE):
            i = g * _FIRE + j
            cps.append(pltpu.async_copy(
                tbl_hbm.at[idx_v.at[i]], vx.at[pl.ds(i * 128, 128)], sem))
            cps.append(pltpu.async_copy(
                tbl_hbm.at[idy_v.at[i]], vy.at[pl.ds(i * 128, 128)], sem))
        for cp in cps:
            cp.wait()

    pl.loop(0, _SLICES // _FIRE)(grp)
    pltpu.sync_copy(vx, ox_hbm.at[pl.ds(base, _ROWS_PER_W)])
    pltpu.sync_copy(vy, oy_hbm.at[pl.ds(base, _ROWS_PER_W)])


def _sc_gather(mesh_1d, gidx2):
    n_pad = gidx2.shape[1] * 128
    mesh = plsc.VectorSubcoreMesh(core_axis_name="c", subcore_axis_name="s")
    k = functools.partial(
        pl.kernel,
        mesh=mesh,
        out_type=[jax.ShapeDtypeStruct((n_pad,), jnp.float32),
                  jax.ShapeDtypeStruct((n_pad,), jnp.float32)],
        scratch_types=[
            pltpu.VMEM((_SLICES, 128), jnp.int32),
            pltpu.VMEM((_SLICES, 128), jnp.int32),
            pltpu.VMEM((_ROWS_PER_W,), jnp.float32),
            pltpu.VMEM((_ROWS_PER_W,), jnp.float32),
            pltpu.SemaphoreType.DMA,
        ],
    )(_sc_gather_body)
    return k(mesh_1d, gidx2)


def _centers_body(mpg_ref, msk_ref, c_ref):
    v = mpg_ref[...]
    m = msk_ref[...]
    lane = lax.broadcasted_iota(jnp.int32, (64, 2), 0)
    col = lax.broadcasted_iota(jnp.int32, (64, 2), 1)
    sel = ((lane % 2) == col).astype(jnp.float32)
    s = jnp.dot(v, sel, preferred_element_type=jnp.float32,
                precision=lax.Precision.HIGHEST)
    den = jnp.sum(m, axis=-1, keepdims=True) + 1e-8
    c_ref[...] = s / den


def _tc_centers(mp_g, cluster_mask):
    B, K, C = cluster_mask.shape
    mpg = mp_g.reshape(B * K, 2 * C)
    msk = cluster_mask.reshape(B * K, C)
    out = pl.pallas_call(
        _centers_body,
        out_shape=jax.ShapeDtypeStruct((B * K, 2), jnp.float32),
    )(mpg, msk)
    return out.reshape(B, K, 2)


def _emb32(p2):
    rows = p2.shape[0]
    x = jnp.broadcast_to(p2[:, 0:1], (rows, 16))
    y = jnp.broadcast_to(p2[:, 1:2], (rows, 16))
    p32 = jnp.concatenate([x, y], axis=-1)
    lane = lax.broadcasted_iota(jnp.int32, (rows, 32), 1)
    freq = (1 << (lane % 8)).astype(jnp.float32) * PI
    phase = jnp.where((lane % 16) >= 8, 0.5 * PI, 0.0)
    return jnp.cos(p32 * freq - phase)


def _pe_body(a_ref, b_ref, o_ref):
    o_ref[...] = _emb32(a_ref[...] - b_ref[...])


def _tc_pe(crep, mp_g_flat):
    n = crep.shape[0]
    rb = 4000
    return pl.pallas_call(
        _pe_body,
        grid=(n // rb,),
        in_specs=[pl.BlockSpec((rb, 2), lambda i: (i, 0)),
                  pl.BlockSpec((rb, 2), lambda i: (i, 0))],
        out_specs=pl.BlockSpec((rb, 32), lambda i: (i, 0)),
        out_shape=jax.ShapeDtypeStruct((n, 32), jnp.float32),
    )(crep, mp_g_flat)


def _out1_body(mesh_ref, rel_ref, o_ref):
    o_ref[...] = jnp.concatenate(
        [_emb32(mesh_ref[...]), rel_ref[...]], axis=-1)


def _tc_out1(mesh_flat, rel_flat):
    n = mesh_flat.shape[0]
    rb = 4000
    return pl.pallas_call(
        _out1_body,
        grid=(n // rb,),
        in_specs=[pl.BlockSpec((rb, 2), lambda i: (i, 0)),
                  pl.BlockSpec((rb, 32), lambda i: (i, 0))],
        out_specs=pl.BlockSpec((rb, 64), lambda i: (i, 0)),
        out_shape=jax.ShapeDtypeStruct((n, 64), jnp.float32),
    )(mesh_flat, rel_flat)


def _out2_body(c_ref, o_ref):
    o_ref[...] = _emb32(c_ref[...])


def _tc_out2(centers_flat):
    n = centers_flat.shape[0]
    return pl.pallas_call(
        _out2_body,
        grid=(1,),
        in_specs=[pl.BlockSpec((n, 2), lambda i: (0, 0))],
        out_specs=pl.BlockSpec((n, 32), lambda i: (0, 0)),
        out_shape=jax.ShapeDtypeStruct((n, 32), jnp.float32),
    )(centers_flat)


def kernel(mesh_pos, clusters, cluster_mask):
    B, N, _ = mesh_pos.shape
    _, K, C = clusters.shape
    idx = clusters.reshape(B, K * C)

    gidx = (idx + (jnp.arange(B, dtype=jnp.int32) * N)[:, None]).reshape(-1)
    n_pad = 32 * _ROWS_PER_W
    gidx_pad = jnp.concatenate(
        [gidx * 2, jnp.zeros((n_pad - gidx.shape[0],), jnp.int32)])
    gidx2 = jnp.stack([gidx_pad, gidx_pad + 1]).reshape(2, 32 * _SLICES, 128)
    mesh_1d = mesh_pos.reshape(2 * B * N)
    gx, gy = _sc_gather(mesh_1d, gidx2)
    mp_g_flat = jnp.stack([gx[:B * N], gy[:B * N]], axis=-1)
    mesh_flat = mesh_pos.reshape(B * N, 2)
    mp_g = mp_g_flat.reshape(B, K * C, 2)

    centers = _tc_centers(mp_g, cluster_mask)

    crep = jnp.repeat(centers, C, axis=1).reshape(B * K * C, 2)
    pe = _tc_pe(crep, mp_g_flat).reshape(B, K * C, 32)

    relative = jax.vmap(lambda base, i, src: base.at[i].set(src))(pe, idx, pe)
    relative = relative[:, :N]

    out1 = _tc_out1(mesh_flat, relative.reshape(B * N, 32)).reshape(B, N, 64)
    out2 = _tc_out2(centers.reshape(B * K, 2)).reshape(B, K, 32)
    return out1, out2
